# baseline (speedup 1.0000x reference)
import functools

import jax
import jax.numpy as jnp
from jax import lax
from jax.experimental import pallas as pl
from jax.experimental.pallas import tpu as pltpu
from jax.experimental.pallas import tpu_sc as plsc

N = 10000
NPAD = 10240
E = 320000
D = 128
NC = 2
NS = 16
NW = NC * NS
EPW = E // NW
RCHUNK = 125
CHUNK = 128
NCHUNK = EPW // RCHUNK
TRASH = NPAD - 8
ACH = 64
ANCH = NCHUNK * CHUNK // ACH
BLK = 8
NBLK = ANCH // BLK
RPT = NPAD // NS
ZROWS = 64
BM = 1280
NB = NPAD // BM
VL = 16


def _mesh():
    return plsc.VectorSubcoreMesh(core_axis_name="c", subcore_axis_name="s")


@functools.partial(
    pl.kernel,
    out_type=jax.ShapeDtypeStruct((NC * NPAD, D), jnp.float32),
    mesh=_mesh(),
    scratch_types=[
        pltpu.VMEM((NCHUNK, CHUNK), jnp.int32),
        pltpu.VMEM((CHUNK, D), jnp.float32),
        pltpu.VMEM((ZROWS, D), jnp.float32),
        pltpu.VMEM_SHARED((NPAD, D), jnp.float32),
        pltpu.SemaphoreType.DMA,
    ],
)
def _deg_kernel(col_hbm, degp_hbm, cidx_v, ones_v, zbuf_v, acc_sh, ssem):
    c = lax.axis_index("c")
    s = lax.axis_index("s")
    wid = s * NC + c

    pltpu.sync_copy(col_hbm.at[wid], cidx_v)

    zeros16 = jnp.zeros((VL,), jnp.float32)
    ones16 = jnp.ones((VL,), jnp.float32)

    def fill_z(i, carry):
        for j in range(D // VL):
            zbuf_v[i, pl.ds(j * VL, VL)] = zeros16
        return carry

    lax.fori_loop(0, ZROWS, fill_z, 0)

    def fill_o(i, carry):
        for j in range(D // VL):
            ones_v[i, pl.ds(j * VL, VL)] = ones16
        return carry

    lax.fori_loop(0, CHUNK, fill_o, 0)

    r0 = s * RPT
    for i in range(RPT // ZROWS):
        pltpu.sync_copy(zbuf_v, acc_sh.at[pl.ds(r0 + i * ZROWS, ZROWS)])
    plsc.subcore_barrier()

    def chunk(j, carry):
        pltpu.async_copy(ones_v, acc_sh.at[cidx_v.at[j]], ssem, add=True)
        return carry

    lax.fori_loop(0, NCHUNK, chunk, 0)

    def drain(j, carry):
        pltpu.make_async_copy(ones_v, acc_sh.at[cidx_v.at[j]], ssem).wait()
        return carry

    lax.fori_loop(0, NCHUNK, drain, 0)
    plsc.subcore_barrier()

    pltpu.sync_copy(acc_sh.at[pl.ds(r0, RPT)],
                    degp_hbm.at[pl.ds(c * NPAD + r0, RPT)])


@functools.partial(
    pl.kernel,
    out_type=jax.ShapeDtypeStruct((NC * NPAD, D), jnp.float32),
    mesh=_mesh(),
    scratch_types=[
        pltpu.VMEM((2, BLK, ACH), jnp.int32),
        pltpu.VMEM((2, BLK, ACH), jnp.int32),
        pltpu.VMEM((ACH, D), jnp.float32),
        pltpu.VMEM((ACH, D), jnp.float32),
        pltpu.VMEM((ACH, D), jnp.float32),
        pltpu.VMEM((ACH, D), jnp.float32),
        pltpu.VMEM((ZROWS, D), jnp.float32),
        pltpu.VMEM_SHARED((NPAD, D), jnp.float32),
        pltpu.SemaphoreType.DMA,
        pltpu.SemaphoreType.DMA,
        pltpu.SemaphoreType.DMA,
        pltpu.SemaphoreType.DMA,
        pltpu.SemaphoreType.DMA,
        pltpu.SemaphoreType.DMA,
        pltpu.SemaphoreType.DMA,
        pltpu.SemaphoreType.DMA,
        pltpu.SemaphoreType.DMA,
        pltpu.SemaphoreType.DMA,
    ],
)
def _agg_kernel(y_hbm, row_hbm, col_hbm, aggp_hbm,
                rblk_v, cblk_v, buf0_v, buf1_v, buf2_v, buf3_v, zbuf_v,
                acc_sh, gsem0, gsem1, gsem2, gsem3,
                ssem0, ssem1, ssem2, ssem3, isem0, isem1):
    c = lax.axis_index("c")
    s = lax.axis_index("s")
    wid = s * NC + c

    bufs = (buf0_v, buf1_v, buf2_v, buf3_v)
    gsems = (gsem0, gsem1, gsem2, gsem3)
    ssems = (ssem0, ssem1, ssem2, ssem3)
    isems = (isem0, isem1)

    def _fetch(B, slot):
        pltpu.async_copy(row_hbm.at[wid, B], rblk_v.at[slot], isems[slot])
        pltpu.async_copy(col_hbm.at[wid, B], cblk_v.at[slot], isems[slot])

    def _wait_fetch(B, slot):
        pltpu.make_async_copy(row_hbm.at[wid, B], rblk_v.at[slot],
                              isems[slot]).wait()
        pltpu.make_async_copy(col_hbm.at[wid, B], cblk_v.at[slot],
                              isems[slot]).wait()

    def _gather(j, bs, islot, irow):
        pltpu.async_copy(y_hbm.at[rblk_v.at[islot, irow]], bufs[bs],
                         gsems[bs])

    def _wait_gather(bs, islot, irow):
        pltpu.make_async_copy(y_hbm.at[rblk_v.at[islot, irow]], bufs[bs],
                              gsems[bs]).wait()

    def _scatter(j, bs, islot, irow):
        pltpu.async_copy(bufs[bs], acc_sh.at[cblk_v.at[islot, irow]],
                         ssems[bs], add=True)

    def _wait_scatter(bs, islot, irow):
        pltpu.make_async_copy(bufs[bs], acc_sh.at[cblk_v.at[islot, irow]],
                              ssems[bs]).wait()

    def _gslot(b):
        return 0 if (b < 6 or b >= 14) else 1

    def _sslot(b):
        return 0 if b < 8 else 1

    pltpu.sync_copy(row_hbm.at[wid, 0], rblk_v.at[0])
    pltpu.sync_copy(col_hbm.at[wid, 0], cblk_v.at[0])
    _gather(0, 0, 0, 0)
    _gather(1, 1, 0, 1)

    zeros16 = jnp.zeros((VL,), jnp.float32)

    def fill_z(i, carry):
        for j in range(D // VL):
            zbuf_v[i, pl.ds(j * VL, VL)] = zeros16
        return carry

    lax.fori_loop(0, ZROWS, fill_z, 0)

    r0 = s * RPT
    for i in range(RPT // ZROWS):
        pltpu.sync_copy(zbuf_v, acc_sh.at[pl.ds(r0 + i * ZROWS, ZROWS)])
    plsc.subcore_barrier()

    def step(g, b, first_group=False, last_group=False):
        j = 16 * g + b
        bs = b % 4
        ns = (b + 2) % 4
        if not (first_group and b < 2):
            _wait_scatter(ns, _sslot((b - 2) % 16), (b - 2) % 8)
        if b == 1:
            _fetch(2 * g + 1, 1)
        if b == 9 and not last_group:
            _fetch(2 * g + 2, 0)
        if b == 6:
            _wait_fetch(2 * g + 1, 1)
        if not (last_group and b >= 14):
            if b == 14 and not last_group:
                _wait_fetch(2 * g + 2, 0)
            _gather(j + 2, ns, _gslot(b), (b + 2) % 8)
        _wait_gather(bs, _sslot(b), b % 8)
        _scatter(j, bs, _sslot(b), b % 8)

    for b in range(16):
        step(0, b, first_group=True)

    def group(g, carry):
        for b in range(16):
            step(g, b)
        return carry

    lax.fori_loop(1, ANCH // 16 - 1, group, 0)

    for b in range(16):
        step(ANCH // 16 - 1, b, last_group=True)
    _wait_scatter(2, 1, 6)
    _wait_scatter(3, 1, 7)
    plsc.subcore_barrier()

    pltpu.sync_copy(acc_sh.at[pl.ds(r0, RPT)],
                    aggp_hbm.at[pl.ds(c * NPAD + r0, RPT)])


def _dis(d0_ref, d1_ref):
    return lax.rsqrt(1.0 + d0_ref[:, 0:1] + d1_ref[:, 0:1])


def _mm_scale_body(x_ref, w_ref, d0_ref, d1_ref, y_ref):
    dis = _dis(d0_ref, d1_ref)
    y_ref[...] = jnp.dot(x_ref[...], w_ref[...],
                         preferred_element_type=jnp.float32) * dis


def _tc_layer1(x, W1, degp):
    return pl.pallas_call(
        _mm_scale_body,
        grid=(NB,),
        in_specs=[
            pl.BlockSpec((BM, D), lambda i: (i, 0)),
            pl.BlockSpec((D, D), lambda i: (0, 0)),
            pl.BlockSpec((BM, D), lambda i: (i, 0)),
            pl.BlockSpec((BM, D), lambda i: (NB + i, 0)),
        ],
        out_specs=pl.BlockSpec((BM, D), lambda i: (i, 0)),
        out_shape=jax.ShapeDtypeStruct((NPAD, D), jnp.float32),
    )(x, W1, degp, degp)


def _combine_mm_body(a0_ref, a1_ref, y1_ref, d0_ref, d1_ref, w_ref, b_ref,
                     y2_ref):
    dis = _dis(d0_ref, d1_ref)
    h = jnp.maximum(
        dis * (a0_ref[...] + a1_ref[...] + y1_ref[...]) + b_ref[...], 0.0)
    y2_ref[...] = jnp.dot(h, w_ref[...],
                          preferred_element_type=jnp.float32) * dis


def _tc_layer2(aggp, y1, degp, W2, b1):
    return pl.pallas_call(
        _combine_mm_body,
        grid=(NB,),
        in_specs=[
            pl.BlockSpec((BM, D), lambda i: (i, 0)),
            pl.BlockSpec((BM, D), lambda i: (NB + i, 0)),
            pl.BlockSpec((BM, D), lambda i: (i, 0)),
            pl.BlockSpec((BM, D), lambda i: (i, 0)),
            pl.BlockSpec((BM, D), lambda i: (NB + i, 0)),
            pl.BlockSpec((D, D), lambda i: (0, 0)),
            pl.BlockSpec((1, D), lambda i: (0, 0)),
        ],
        out_specs=pl.BlockSpec((BM, D), lambda i: (i, 0)),
        out_shape=jax.ShapeDtypeStruct((NPAD, D), jnp.float32),
    )(aggp, aggp, y1, degp, degp, W2, b1)


def _final_body(a0_ref, a1_ref, y2_ref, d0_ref, d1_ref, b_ref, o_ref):
    dis = _dis(d0_ref, d1_ref)
    o_ref[...] = jnp.maximum(
        dis * (a0_ref[...] + a1_ref[...] + y2_ref[...]) + b_ref[...], 0.0)


def _tc_final(aggp, y2, degp, b2):
    return pl.pallas_call(
        _final_body,
        grid=(NB,),
        in_specs=[
            pl.BlockSpec((BM, D), lambda i: (i, 0)),
            pl.BlockSpec((BM, D), lambda i: (NB + i, 0)),
            pl.BlockSpec((BM, D), lambda i: (i, 0)),
            pl.BlockSpec((BM, D), lambda i: (i, 0)),
            pl.BlockSpec((BM, D), lambda i: (NB + i, 0)),
            pl.BlockSpec((1, D), lambda i: (0, 0)),
        ],
        out_specs=pl.BlockSpec((BM, D), lambda i: (i, 0)),
        out_shape=jax.ShapeDtypeStruct((NPAD, D), jnp.float32),
    )(aggp, aggp, y2, degp, degp, b2)


def kernel(x, edge_index, W1, b1, W2, b2):
    ei = edge_index.astype(jnp.int32)
    row = ei[0].reshape(NW, NCHUNK, RCHUNK)
    col = ei[1].reshape(NW, NCHUNK, RCHUNK)
    row = jnp.pad(row, ((0, 0), (0, 0), (0, CHUNK - RCHUNK)))
    col = jnp.pad(col, ((0, 0), (0, 0), (0, CHUNK - RCHUNK)),
                  constant_values=TRASH)
    row4 = row.reshape(NW, NBLK, BLK, ACH)
    col4 = col.reshape(NW, NBLK, BLK, ACH)
    xp = jnp.pad(x, ((0, NPAD - N), (0, 0)))
    degp = _deg_kernel(col)
    y1 = _tc_layer1(xp, W1, degp)
    a1 = _agg_kernel(y1, row4, col4)
    y2 = _tc_layer2(a1, y1, degp, W2, b1.reshape(1, D))
    a2 = _agg_kernel(y2, row4, col4)
    return _tc_final(a2, y2, degp, b2.reshape(1, D))[:N]
